# baseline (speedup 1.0000x reference)
import jax
import jax.numpy as jnp
from jax.experimental import pallas as pl

_EGO = 0.01831563888873418
_EGE = 0.36787944117144233
_NEG = -1e30
_B = 16
_L = 256
_CHUNK = 32
_NCHUNK = 16


def _shiftn(x):
    return jnp.concatenate(
        [jnp.full((x.shape[0], 1), _NEG, x.dtype), x[:, :-1]], axis=1)


def _shift0(x):
    return jnp.concatenate(
        [jnp.zeros((x.shape[0], 1), x.dtype), x[:, :-1]], axis=1)


def _shift1f(x):
    return jnp.concatenate(
        [jnp.ones((x.shape[0], 1), x.dtype), x[:, :-1]], axis=1)


def _rotk(x, k):
    k = k % _L
    if k == 0:
        return x
    return jnp.concatenate([x[:, -k:], x[:, :-k]], axis=1)


def _sel4(t, v):
    return jnp.where(t == 0, v[0],
           jnp.where(t == 1, v[1],
           jnp.where(t == 2, v[2], v[3])))


def _sw_kernel(predT_ref, v0_ref, out_ref):
    predT = predT_ref[...]
    v = v0_ref[...]
    lane = jax.lax.broadcasted_iota(jnp.int32, (_B, _L), 1)
    vmask = lane < (_L - 1)
    zero = jnp.zeros((_B, _L), jnp.float32)

    p0p = [jnp.where(vmask, jnp.maximum(predT[p], 0.0), zero) for p in range(4)]
    ep1 = [jnp.where(vmask,
                     jnp.exp(jnp.concatenate(
                         [predT[p][:, 1:], predT[p][:, :1]], axis=1)),
                     zero) for p in range(4)]

    def chunk(i, carry):
        (vc, mx1, shs1, shs2, ea1, er1, ssq1, seg1, seg2, eg1, acc,
         e01, dqp, w1) = carry
        d_base = i * _CHUNK
        ld0 = lane - d_base
        ws = [_rotk(vc, k) for k in range(_CHUNK + 1)]
        for k in range(_CHUNK):
            ld = ld0 - k
            mask = (ld <= 0) & (ld >= -254)
            sp = jnp.where(mask, _sel4(ws[k], p0p), zero)
            esmx = jnp.where(mask, jnp.exp(sp), zero)
            exe = jnp.where(mask, _sel4(ws[k + 1], ep1), zero)
            mx0 = jnp.maximum(jnp.maximum(shs2 + sp, shs1), mx1)
            d1 = jnp.exp(mx1 - mx0)
            dq = jnp.exp(shs1 - mx0)
            dg = dqp * d1
            e00 = e01 * d1
            ea0 = esmx * (seg2 * dg + e00)
            er0 = (w1 + _EGE * er1) * d1
            ed0 = ssq1 * dq
            eg0 = ea0 + er0 + ed0
            w0 = _EGO * ea0
            sq0 = w0 + _EGO * er0 + _EGE * ed0
            acc = acc * d1 + eg0 * exe
            shs2 = shs1
            shs1 = _shiftn(mx0)
            seg2 = seg1
            seg1 = _shift0(eg0)
            ssq1 = _shift0(sq0)
            mx1 = mx0
            ea1, er1 = ea0, er0
            dqp = dq
            e01 = e00
            eg1 = eg0
            w1 = w0
        vc = ws[_CHUNK]
        n1 = jnp.maximum(eg1, 1.0)
        r1 = 1.0 / n1
        mx1 = mx1 + jnp.log(n1)
        ea1 = ea1 * r1
        er1 = er1 * r1
        eg1 = eg1 * r1
        acc = acc * r1
        e01 = e01 * r1
        w1 = w1 * r1
        sr1 = _shift1f(r1)
        ssq1 = ssq1 * sr1
        seg1 = seg1 * sr1
        shs1 = shs1 + jnp.log(_shift1f(n1))
        n2s = jnp.maximum(seg2, 1.0)
        seg2 = seg2 / n2s
        shs2 = shs2 + jnp.log(n2s)
        dqp = dqp * n2s * r1
        return (vc, mx1, shs1, shs2, ea1, er1, ssq1, seg1, seg2, eg1, acc,
                e01, dqp, w1)

    zi = predT[0] * 0.0
    negi = zi + _NEG
    onei = zi + 1.0
    init = (v, zi, negi, negi, zi, zi, zi, zi, zi, zi, zi, onei, zi, zi)
    out = jax.lax.fori_loop(0, _NCHUNK, chunk, init)
    mx1, acc = out[1], out[10]

    t = mx1 + jnp.log(jnp.maximum(acc, 1e-35))
    mb = jnp.max(t, axis=1, keepdims=True)
    sb = jnp.sum(jnp.exp(t - mb), axis=1, keepdims=True)
    fin = mb + jnp.log(sb)
    out_ref[...] = jnp.full((1, 1), -jnp.sum(fin) * (1.0 / _B), jnp.float32)


def _prep(predictions, targets):
    predT = jnp.transpose(predictions.astype(jnp.float32), (2, 0, 1))
    t = targets.astype(jnp.int32)
    v0 = jnp.concatenate([t[:, :1], jnp.flip(t[:, 1:], axis=1)], axis=1)
    return predT, v0


@jax.jit
def kernel(predictions, targets):
    predT, v0 = _prep(predictions, targets)
    out = pl.pallas_call(
        _sw_kernel,
        out_shape=jax.ShapeDtypeStruct((1, 1), jnp.float32),
    )(predT, v0)
    return out[0, 0]
